# baseline (speedup 1.0000x reference)
import functools

import jax
import jax.numpy as jnp
from jax import lax
from jax.experimental import pallas as pl
from jax.experimental.pallas import tpu as pltpu
from jax.experimental.pallas import tpu_sc as plsc

N = 10000
NP = 10240
E = 320000
EP = 327680
ES = 100000
ESP = 102400
H = 128
HH = 64
DE = 16
CE = 128
NCHUNK = EP // 32 // CE
NH = 5120
NR = 5632
STRIPE = NR // 16
PE = ESP // 32

_mesh = plsc.VectorSubcoreMesh(core_axis_name="c", subcore_axis_name="s")
_sc_params = pltpu.CompilerParams(needs_layout_passes=False)


def _mm_bias_body(x_ref, w_ref, b_ref, o_ref):
    o_ref[...] = jnp.dot(x_ref[...], w_ref[...],
                         preferred_element_type=jnp.float32) + b_ref[...]


def _mm_bias(x, w, b, blocks):
    m, k = x.shape
    ko = w.shape[1]
    bsz = m // blocks
    return pl.pallas_call(
        _mm_bias_body,
        grid=(blocks,),
        in_specs=[pl.BlockSpec((bsz, k), lambda i: (i, 0)),
                  pl.BlockSpec((k, ko), lambda i: (0, 0)),
                  pl.BlockSpec((1, ko), lambda i: (0, 0))],
        out_specs=pl.BlockSpec((bsz, ko), lambda i: (i, 0)),
        out_shape=jax.ShapeDtypeStruct((m, ko), jnp.float32),
    )(x, w, b)


def _mm_body(x_ref, w_ref, o_ref):
    o_ref[...] = jnp.dot(x_ref[...], w_ref[...],
                         preferred_element_type=jnp.float32)


def _mm(x, w, blocks):
    m, k = x.shape
    ko = w.shape[1]
    bsz = m // blocks
    return pl.pallas_call(
        _mm_body,
        grid=(blocks,),
        in_specs=[pl.BlockSpec((bsz, k), lambda i: (i, 0)),
                  pl.BlockSpec((k, ko), lambda i: (0, 0))],
        out_specs=pl.BlockSpec((bsz, ko), lambda i: (i, 0)),
        out_shape=jax.ShapeDtypeStruct((m, ko), jnp.float32),
    )(x, w)


def _tables_body(h_ref, d_ref, b_ref, ew_ref, ts_ref, te_ref):
    h = h_ref[...]
    ts_ref[:, :H] = jnp.dot(h, d_ref[...], preferred_element_type=jnp.float32)
    ts_ref[:, H:] = jnp.dot(h, b_ref[...], preferred_element_type=jnp.float32)
    te_ref[...] = jnp.dot(h, ew_ref[...], preferred_element_type=jnp.float32)


def _tables(h, dl, bl, ewl, blocks=10):
    bsz = NP // blocks
    return pl.pallas_call(
        _tables_body,
        grid=(blocks,),
        in_specs=[pl.BlockSpec((bsz, H), lambda i: (i, 0)),
                  pl.BlockSpec((H, H), lambda i: (0, 0)),
                  pl.BlockSpec((H, H), lambda i: (0, 0)),
                  pl.BlockSpec((H, H), lambda i: (0, 0))],
        out_specs=[
            pl.BlockSpec((bsz, 2 * H), lambda i: (i, 0)),
            pl.BlockSpec((bsz, H), lambda i: (i, 0)),
        ],
        out_shape=[jax.ShapeDtypeStruct((NP, 2 * H), jnp.float32),
                   jax.ShapeDtypeStruct((NP, H), jnp.float32)],
    )(h, dl, bl, ewl)


def _hupd_body(h_ref, a_ref, agg_ref, den_ref, o_ref):
    h = h_ref[...]
    ha = jnp.dot(h, a_ref[...], preferred_element_type=jnp.float32)
    agg = agg_ref[0, 0] + agg_ref[0, 1]
    den = den_ref[0, 0] + den_ref[0, 1]
    o_ref[...] = jnp.maximum(ha + agg / (den + 1e-6), 0.0) + h


def _h_update(h, al, agg4, den4):
    bsz = 512
    nb = NH // bsz
    return pl.pallas_call(
        _hupd_body,
        grid=(NP // bsz,),
        in_specs=[pl.BlockSpec((bsz, H), lambda i: (i, 0)),
                  pl.BlockSpec((H, H), lambda i: (0, 0)),
                  pl.BlockSpec((1, 2, bsz, H),
                               lambda i: (i // nb, 0, i % nb, 0)),
                  pl.BlockSpec((1, 2, bsz, H),
                               lambda i: (i // nb, 0, i % nb, 0))],
        out_specs=pl.BlockSpec((bsz, H), lambda i: (i, 0)),
        out_shape=jax.ShapeDtypeStruct((NP, H), jnp.float32),
    )(h, al, agg4, den4)


def _sc_edge_pass(src, dst, p, ef, ts, te):
    @functools.partial(
        pl.kernel,
        out_type=[
            jax.ShapeDtypeStruct((EP, H), jnp.float32),
            jax.ShapeDtypeStruct((2, EP, H), jnp.float32),
            jax.ShapeDtypeStruct((4, 2, NR, H), jnp.float32),
        ],
        mesh=_mesh,
        scratch_types=[
            pltpu.VMEM((CE,), jnp.int32),
            pltpu.VMEM((CE,), jnp.int32),
            pltpu.VMEM((CE,), jnp.int32),
            pltpu.VMEM((CE, 2 * H), jnp.float32),
            pltpu.VMEM((CE, H), jnp.float32),
            pltpu.VMEM((CE, H), jnp.float32),
            pltpu.VMEM((CE, H), jnp.float32),
            pltpu.VMEM_SHARED((NR, H), jnp.float32),
        ],
        compiler_params=_sc_params,
    )
    def k(src_hbm, dst_hbm, p_hbm, ef_hbm, ts_hbm, te_hbm,
          oef_hbm, val_hbm, acc_hbm,
          si_v, di_v, ci_v, ts_v, te_v, p_v, sg_v, acc_sp):
        c = lax.axis_index("c")
        s = lax.axis_index("s")
        ebase = (c * 16 + s) * (EP // 32)
        zb = s * STRIPE

        @pl.loop(0, NCHUNK)
        def _(kk):
            base = ebase + kk * CE
            pltpu.sync_copy(dst_hbm.at[pl.ds(base, CE)], di_v)
            pltpu.sync_copy(src_hbm.at[pl.ds(base, CE)], si_v)
            pltpu.sync_copy(ts_hbm.at[si_v], ts_v)
            pltpu.sync_copy(te_hbm.at[di_v], te_v)
            pltpu.sync_copy(p_hbm.at[pl.ds(base, CE)], p_v)

            @pl.loop(0, CE)
            def _(r):
                @pl.loop(0, H, step=16)
                def _(j):
                    dsl = pl.ds(j, 16)
                    eh = p_v[r, dsl] + ts_v[r, dsl] + te_v[r, dsl]
                    sg = 1.0 / (1.0 + jnp.exp(-eh))
                    ts_v[r, pl.ds(H + j, 16)] = sg * ts_v[r, pl.ds(H + j, 16)]
                    te_v[r, dsl] = eh
                    sg_v[r, dsl] = sg

            pltpu.sync_copy(ts_v.at[:, pl.ds(H, H)],
                            val_hbm.at[0, pl.ds(base, CE)])
            pltpu.sync_copy(sg_v, val_hbm.at[1, pl.ds(base, CE)])
            pltpu.sync_copy(ef_hbm.at[pl.ds(base, CE)],
                            ts_v.at[:, pl.ds(0, H)])

            @pl.loop(0, CE)
            def _(r):
                @pl.loop(0, H, step=16)
                def _(j):
                    dsl = pl.ds(j, 16)
                    p_v[r, dsl] = (jnp.maximum(te_v[r, dsl], 0.0)
                                   + ts_v[r, dsl])

            pltpu.sync_copy(p_v, oef_hbm.at[pl.ds(base, CE)])

        @pl.loop(0, 4)
        def _(ph):
            q = ph // 2
            lo = (ph % 2) * NH

            @pl.loop(0, CE)
            def _(r):
                @pl.loop(0, H, step=16)
                def _(j):
                    te_v[r, pl.ds(j, 16)] = jnp.zeros((16,), jnp.float32)

            pltpu.sync_copy(te_v, acc_sp.at[pl.ds(zb, CE)])
            pltpu.sync_copy(te_v, acc_sp.at[pl.ds(zb + CE, CE)])
            pltpu.sync_copy(te_v.at[pl.ds(0, STRIPE - 2 * CE)],
                            acc_sp.at[pl.ds(zb + 2 * CE, STRIPE - 2 * CE)])
            plsc.subcore_barrier()

            @pl.loop(0, NCHUNK)
            def _(kk):
                base = ebase + kk * CE
                pltpu.sync_copy(dst_hbm.at[pl.ds(base, CE)], di_v)
                pltpu.sync_copy(val_hbm.at[q, pl.ds(base, CE)], sg_v)

                @pl.loop(0, CE, step=16)
                def _(i):
                    d = di_v[pl.ds(i, 16)] - lo
                    ok = (d >= 0) & (d < NH)
                    ci_v[pl.ds(i, 16)] = jnp.where(ok, d, NH)

                pltpu.sync_copy(sg_v, acc_sp.at[ci_v], add=True)

            plsc.subcore_barrier()
            pltpu.sync_copy(acc_sp.at[pl.ds(zb, STRIPE)],
                            acc_hbm.at[ph, c, pl.ds(zb, STRIPE)])
            plsc.subcore_barrier()

    return k(src, dst, p, ef, ts, te)


def _sc_predict(s2, d2, q, u, v):
    @functools.partial(
        pl.kernel,
        out_type=jax.ShapeDtypeStruct((ESP,), jnp.float32),
        mesh=_mesh,
        scratch_types=[
            pltpu.VMEM((NP,), jnp.float32),
            pltpu.VMEM((NP,), jnp.float32),
            pltpu.VMEM((PE,), jnp.int32),
            pltpu.VMEM((PE,), jnp.int32),
            pltpu.VMEM((PE,), jnp.float32),
        ],
        compiler_params=_sc_params,
    )
    def k(s2_hbm, d2_hbm, q_hbm, u_hbm, v_hbm, o_hbm,
          u_v, v_v, si_v, di_v, q_v):
        c = lax.axis_index("c")
        s = lax.axis_index("s")
        base = (c * 16 + s) * PE
        pltpu.sync_copy(u_hbm, u_v)
        pltpu.sync_copy(v_hbm, v_v)
        pltpu.sync_copy(s2_hbm.at[pl.ds(base, PE)], si_v)
        pltpu.sync_copy(d2_hbm.at[pl.ds(base, PE)], di_v)
        pltpu.sync_copy(q_hbm.at[pl.ds(base, PE)], q_v)

        @pl.loop(0, PE, step=16)
        def _(i):
            dsl = pl.ds(i, 16)
            ug = plsc.load_gather(u_v, [si_v[dsl]])
            vg = plsc.load_gather(v_v, [di_v[dsl]])
            q_v[dsl] = q_v[dsl] + ug + vg

        pltpu.sync_copy(q_v, o_hbm.at[pl.ds(base, PE)])

    return k(s2, d2, q, u, v)


def kernel(edge_index_sub, edge_index, x, e, e_subgraph,
           Wn, bn, We, be, A, B, C, D, Ew, Wp, bp):
    f32 = jnp.float32
    src = jnp.pad(edge_index[0].astype(jnp.int32), (0, EP - E))
    dst = jnp.pad(edge_index[1].astype(jnp.int32), (0, EP - E),
                  constant_values=N)
    s2 = jnp.pad(edge_index_sub[0].astype(jnp.int32), (0, ESP - ES))
    d2 = jnp.pad(edge_index_sub[1].astype(jnp.int32), (0, ESP - ES))

    xp = jnp.pad(x.astype(f32), ((0, NP - N), (0, 0)))
    ep = jnp.pad(e.astype(f32), ((0, EP - E), (0, 0)))
    esp = jnp.pad(e_subgraph.astype(f32), ((0, ESP - ES), (0, 0)))

    h = _mm_bias(xp, Wn, bn.reshape(1, H), blocks=10)
    ef = _mm_bias(ep, We, be.reshape(1, H), blocks=40)

    for l in range(A.shape[0]):
        ts, te = _tables(h, D[l], B[l], Ew[l])
        p = _mm(ef, C[l], blocks=40)
        ef, _, acc = _sc_edge_pass(src, dst, p, ef, ts, te)
        h = _h_update(h, A[l], acc[:2], acc[2:])

    wp12 = jnp.concatenate([Wp[:H], Wp[H:2 * H]], axis=1)
    w3 = We @ Wp[2 * H:]
    cst = (be @ Wp[2 * H:] + bp).reshape(1, 1)

    uv = _mm(h, wp12, blocks=10)
    q = _mm_bias(esp, w3, cst, blocks=25).reshape(ESP)
    scores = _sc_predict(s2, d2, q, uv[:, 0], uv[:, 1])
    return scores[:ES].reshape(ES, 1)


# speedup vs baseline: 1.1705x; 1.1705x over previous
import functools

import jax
import jax.numpy as jnp
from jax import lax
from jax.experimental import pallas as pl
from jax.experimental.pallas import tpu as pltpu
from jax.experimental.pallas import tpu_sc as plsc

N = 10000
NP = 10240
E = 320000
EP = 327680
ES = 100000
ESP = 102400
H = 128
HH = 64
DE = 16
CE = 128
NCHUNK = EP // 32 // CE
NH = 5120
NR = 5632
STRIPE = NR // 16
PE = ESP // 32

_mesh = plsc.VectorSubcoreMesh(core_axis_name="c", subcore_axis_name="s")
_sc_params = pltpu.CompilerParams(needs_layout_passes=False)


def _mm_bias_body(x_ref, w_ref, b_ref, o_ref):
    o_ref[...] = jnp.dot(x_ref[...], w_ref[...],
                         preferred_element_type=jnp.float32) + b_ref[...]


def _mm_bias(x, w, b, blocks):
    m, k = x.shape
    ko = w.shape[1]
    bsz = m // blocks
    return pl.pallas_call(
        _mm_bias_body,
        grid=(blocks,),
        in_specs=[pl.BlockSpec((bsz, k), lambda i: (i, 0)),
                  pl.BlockSpec((k, ko), lambda i: (0, 0)),
                  pl.BlockSpec((1, ko), lambda i: (0, 0))],
        out_specs=pl.BlockSpec((bsz, ko), lambda i: (i, 0)),
        out_shape=jax.ShapeDtypeStruct((m, ko), jnp.float32),
    )(x, w, b)


def _mm_body(x_ref, w_ref, o_ref):
    o_ref[...] = jnp.dot(x_ref[...], w_ref[...],
                         preferred_element_type=jnp.float32)


def _mm(x, w, blocks):
    m, k = x.shape
    ko = w.shape[1]
    bsz = m // blocks
    return pl.pallas_call(
        _mm_body,
        grid=(blocks,),
        in_specs=[pl.BlockSpec((bsz, k), lambda i: (i, 0)),
                  pl.BlockSpec((k, ko), lambda i: (0, 0))],
        out_specs=pl.BlockSpec((bsz, ko), lambda i: (i, 0)),
        out_shape=jax.ShapeDtypeStruct((m, ko), jnp.float32),
    )(x, w)


def _tables_body(h_ref, d_ref, b_ref, ew_ref, ts_ref, te_ref):
    h = h_ref[...]
    ts_ref[:, :H] = jnp.dot(h, d_ref[...], preferred_element_type=jnp.float32)
    ts_ref[:, H:] = jnp.dot(h, b_ref[...], preferred_element_type=jnp.float32)
    te_ref[...] = jnp.dot(h, ew_ref[...], preferred_element_type=jnp.float32)


def _tables(h, dl, bl, ewl, blocks=10):
    bsz = NP // blocks
    return pl.pallas_call(
        _tables_body,
        grid=(blocks,),
        in_specs=[pl.BlockSpec((bsz, H), lambda i: (i, 0)),
                  pl.BlockSpec((H, H), lambda i: (0, 0)),
                  pl.BlockSpec((H, H), lambda i: (0, 0)),
                  pl.BlockSpec((H, H), lambda i: (0, 0))],
        out_specs=[
            pl.BlockSpec((bsz, 2 * H), lambda i: (i, 0)),
            pl.BlockSpec((bsz, H), lambda i: (i, 0)),
        ],
        out_shape=[jax.ShapeDtypeStruct((NP, 2 * H), jnp.float32),
                   jax.ShapeDtypeStruct((NP, H), jnp.float32)],
    )(h, dl, bl, ewl)


def _hupd_body(h_ref, a_ref, agg_ref, den_ref, o_ref):
    h = h_ref[...]
    ha = jnp.dot(h, a_ref[...], preferred_element_type=jnp.float32)
    agg = agg_ref[0, 0] + agg_ref[0, 1]
    den = den_ref[0, 0] + den_ref[0, 1]
    o_ref[...] = jnp.maximum(ha + agg / (den + 1e-6), 0.0) + h


def _h_update(h, al, agg4, den4):
    bsz = 512
    nb = NH // bsz
    return pl.pallas_call(
        _hupd_body,
        grid=(NP // bsz,),
        in_specs=[pl.BlockSpec((bsz, H), lambda i: (i, 0)),
                  pl.BlockSpec((H, H), lambda i: (0, 0)),
                  pl.BlockSpec((1, 2, bsz, H),
                               lambda i: (i // nb, 0, i % nb, 0)),
                  pl.BlockSpec((1, 2, bsz, H),
                               lambda i: (i // nb, 0, i % nb, 0))],
        out_specs=pl.BlockSpec((bsz, H), lambda i: (i, 0)),
        out_shape=jax.ShapeDtypeStruct((NP, H), jnp.float32),
    )(h, al, agg4, den4)


def _sc_edge_pass(src, dst, p, ef, ts, te):
    @functools.partial(
        pl.kernel,
        out_type=[
            jax.ShapeDtypeStruct((EP, H), jnp.float32),
            jax.ShapeDtypeStruct((2, EP, H), jnp.float32),
            jax.ShapeDtypeStruct((4, 2, NR, H), jnp.float32),
        ],
        mesh=_mesh,
        scratch_types=[
            pltpu.VMEM((CE,), jnp.int32),
            pltpu.VMEM((CE,), jnp.int32),
            pltpu.VMEM((CE,), jnp.int32),
            pltpu.VMEM((CE, 2 * H), jnp.float32),
            pltpu.VMEM((CE, H), jnp.float32),
            pltpu.VMEM((CE, H), jnp.float32),
            pltpu.VMEM((CE, H), jnp.float32),
            pltpu.VMEM_SHARED((NR, H), jnp.float32),
            pltpu.SemaphoreType.DMA,
            pltpu.SemaphoreType.DMA,
        ],
        compiler_params=_sc_params,
    )
    def k(src_hbm, dst_hbm, p_hbm, ef_hbm, ts_hbm, te_hbm,
          oef_hbm, val_hbm, acc_hbm,
          si_v, di_v, ci_v, ts_v, te_v, p_v, sg_v, acc_sp, sem0, sem1):
        c = lax.axis_index("c")
        s = lax.axis_index("s")
        ebase = (c * 16 + s) * (EP // 32)
        zb = s * STRIPE

        def zero_acc():
            @pl.loop(0, CE)
            def _(r):
                @pl.loop(0, H, step=16)
                def _(j):
                    te_v[r, pl.ds(j, 16)] = jnp.zeros((16,), jnp.float32)

            pltpu.sync_copy(te_v, acc_sp.at[pl.ds(zb, CE)])
            pltpu.sync_copy(te_v, acc_sp.at[pl.ds(zb + CE, CE)])
            pltpu.sync_copy(te_v.at[pl.ds(0, STRIPE - 2 * CE)],
                            acc_sp.at[pl.ds(zb + 2 * CE, STRIPE - 2 * CE)])
            plsc.subcore_barrier()

        def readout(dst_slot):
            plsc.subcore_barrier()
            pltpu.sync_copy(acc_sp.at[pl.ds(zb, STRIPE)],
                            acc_hbm.at[dst_slot, c, pl.ds(zb, STRIPE)])
            plsc.subcore_barrier()

        zero_acc()

        @pl.loop(0, NCHUNK)
        def _(kk):
            base = ebase + kk * CE
            d1 = pltpu.async_copy(dst_hbm.at[pl.ds(base, CE)], di_v, sem0)
            d2 = pltpu.async_copy(src_hbm.at[pl.ds(base, CE)], si_v, sem0)
            d1.wait()
            d2.wait()
            g1 = pltpu.async_copy(ts_hbm.at[si_v], ts_v, sem1)
            g2 = pltpu.async_copy(te_hbm.at[di_v], te_v, sem1)
            g3 = pltpu.async_copy(p_hbm.at[pl.ds(base, CE)], p_v, sem1)
            g1.wait()
            g2.wait()
            g3.wait()

            @pl.loop(0, CE)
            def _(r):
                @pl.loop(0, H, step=16)
                def _(j):
                    dsl = pl.ds(j, 16)
                    eh = p_v[r, dsl] + ts_v[r, dsl] + te_v[r, dsl]
                    sg = 1.0 / (1.0 + jnp.exp(-eh))
                    sg_v[r, dsl] = sg * ts_v[r, pl.ds(H + j, 16)]
                    ts_v[r, pl.ds(H + j, 16)] = sg
                    te_v[r, dsl] = eh

            @pl.loop(0, CE, step=16)
            def _(i):
                d = di_v[pl.ds(i, 16)]
                ci_v[pl.ds(i, 16)] = jnp.where(d < NH, d, NH)

            w1 = pltpu.async_copy(sg_v, val_hbm.at[0, pl.ds(base, CE)], sem0)
            w2 = pltpu.async_copy(ts_v.at[:, pl.ds(H, H)],
                                  val_hbm.at[1, pl.ds(base, CE)], sem0)
            w3 = pltpu.async_copy(ef_hbm.at[pl.ds(base, CE)],
                                  ts_v.at[:, pl.ds(0, H)], sem1)
            pltpu.sync_copy(sg_v, acc_sp.at[ci_v], add=True)
            w1.wait()
            w2.wait()
            w3.wait()

            @pl.loop(0, CE)
            def _(r):
                @pl.loop(0, H, step=16)
                def _(j):
                    dsl = pl.ds(j, 16)
                    p_v[r, dsl] = (jnp.maximum(te_v[r, dsl], 0.0)
                                   + ts_v[r, dsl])

            pltpu.sync_copy(p_v, oef_hbm.at[pl.ds(base, CE)])

        readout(0)

        @pl.loop(0, 3)
        def _(ph):
            q = jnp.where(ph == 0, 0, 1)
            lo = jnp.where(ph == 1, 0, NH)
            zero_acc()

            @pl.loop(0, NCHUNK)
            def _(kk):
                base = ebase + kk * CE
                d1 = pltpu.async_copy(dst_hbm.at[pl.ds(base, CE)], di_v,
                                      sem0)
                d2 = pltpu.async_copy(val_hbm.at[q, pl.ds(base, CE)], sg_v,
                                      sem1)
                d1.wait()

                @pl.loop(0, CE, step=16)
                def _(i):
                    d = di_v[pl.ds(i, 16)] - lo
                    ok = (d >= 0) & (d < NH)
                    ci_v[pl.ds(i, 16)] = jnp.where(ok, d, NH)

                d2.wait()
                pltpu.sync_copy(sg_v, acc_sp.at[ci_v], add=True)

            readout(ph + 1)

    return k(src, dst, p, ef, ts, te)


def _sc_predict(s2, d2, q, u, v):
    @functools.partial(
        pl.kernel,
        out_type=jax.ShapeDtypeStruct((ESP,), jnp.float32),
        mesh=_mesh,
        scratch_types=[
            pltpu.VMEM((NP,), jnp.float32),
            pltpu.VMEM((NP,), jnp.float32),
            pltpu.VMEM((PE,), jnp.int32),
            pltpu.VMEM((PE,), jnp.int32),
            pltpu.VMEM((PE,), jnp.float32),
        ],
        compiler_params=_sc_params,
    )
    def k(s2_hbm, d2_hbm, q_hbm, u_hbm, v_hbm, o_hbm,
          u_v, v_v, si_v, di_v, q_v):
        c = lax.axis_index("c")
        s = lax.axis_index("s")
        base = (c * 16 + s) * PE
        pltpu.sync_copy(u_hbm, u_v)
        pltpu.sync_copy(v_hbm, v_v)
        pltpu.sync_copy(s2_hbm.at[pl.ds(base, PE)], si_v)
        pltpu.sync_copy(d2_hbm.at[pl.ds(base, PE)], di_v)
        pltpu.sync_copy(q_hbm.at[pl.ds(base, PE)], q_v)

        @pl.loop(0, PE, step=16)
        def _(i):
            dsl = pl.ds(i, 16)
            ug = plsc.load_gather(u_v, [si_v[dsl]])
            vg = plsc.load_gather(v_v, [di_v[dsl]])
            q_v[dsl] = q_v[dsl] + ug + vg

        pltpu.sync_copy(q_v, o_hbm.at[pl.ds(base, PE)])

    return k(s2, d2, q, u, v)


def kernel(edge_index_sub, edge_index, x, e, e_subgraph,
           Wn, bn, We, be, A, B, C, D, Ew, Wp, bp):
    f32 = jnp.float32
    src = jnp.pad(edge_index[0].astype(jnp.int32), (0, EP - E))
    dst = jnp.pad(edge_index[1].astype(jnp.int32), (0, EP - E),
                  constant_values=N)
    s2 = jnp.pad(edge_index_sub[0].astype(jnp.int32), (0, ESP - ES))
    d2 = jnp.pad(edge_index_sub[1].astype(jnp.int32), (0, ESP - ES))

    xp = jnp.pad(x.astype(f32), ((0, NP - N), (0, 0)))
    ep = jnp.pad(e.astype(f32), ((0, EP - E), (0, 0)))
    esp = jnp.pad(e_subgraph.astype(f32), ((0, ESP - ES), (0, 0)))

    h = _mm_bias(xp, Wn, bn.reshape(1, H), blocks=10)
    ef = _mm_bias(ep, We, be.reshape(1, H), blocks=40)

    for l in range(A.shape[0]):
        ts, te = _tables(h, D[l], B[l], Ew[l])
        p = _mm(ef, C[l], blocks=40)
        ef, _, acc = _sc_edge_pass(src, dst, p, ef, ts, te)
        h = _h_update(h, A[l], acc[:2], acc[2:])

    wp12 = jnp.concatenate([Wp[:H], Wp[H:2 * H]], axis=1)
    w3 = We @ Wp[2 * H:]
    cst = (be @ Wp[2 * H:] + bp).reshape(1, 1)

    uv = _mm(h, wp12, blocks=10)
    q = _mm_bias(esp, w3, cst, blocks=25).reshape(ESP)
    scores = _sc_predict(s2, d2, q, uv[:, 0], uv[:, 1])
    return scores[:ES].reshape(ES, 1)


# speedup vs baseline: 1.3268x; 1.1335x over previous
import functools

import jax
import jax.numpy as jnp
from jax import lax
from jax.experimental import pallas as pl
from jax.experimental.pallas import tpu as pltpu
from jax.experimental.pallas import tpu_sc as plsc

N = 10000
NP = 10240
E = 320000
EP = 327680
ES = 100000
ESP = 102400
H = 128
HH = 64
DE = 16
CE = 128
NCHUNK = EP // 32 // CE
NH = 5120
NR = 5632
STRIPE = NR // 16
PE = ESP // 32

_mesh = plsc.VectorSubcoreMesh(core_axis_name="c", subcore_axis_name="s")
_sc_params = pltpu.CompilerParams(needs_layout_passes=False)


def _mm_bias_body(x_ref, w_ref, b_ref, o_ref):
    o_ref[...] = jnp.dot(x_ref[...], w_ref[...],
                         preferred_element_type=jnp.float32) + b_ref[...]


def _mm_bias(x, w, b, blocks):
    m, k = x.shape
    ko = w.shape[1]
    bsz = m // blocks
    return pl.pallas_call(
        _mm_bias_body,
        grid=(blocks,),
        in_specs=[pl.BlockSpec((bsz, k), lambda i: (i, 0)),
                  pl.BlockSpec((k, ko), lambda i: (0, 0)),
                  pl.BlockSpec((1, ko), lambda i: (0, 0))],
        out_specs=pl.BlockSpec((bsz, ko), lambda i: (i, 0)),
        out_shape=jax.ShapeDtypeStruct((m, ko), jnp.float32),
    )(x, w, b)


def _mm_body(x_ref, w_ref, o_ref):
    o_ref[...] = jnp.dot(x_ref[...], w_ref[...],
                         preferred_element_type=jnp.float32)


def _mm(x, w, blocks):
    m, k = x.shape
    ko = w.shape[1]
    bsz = m // blocks
    return pl.pallas_call(
        _mm_body,
        grid=(blocks,),
        in_specs=[pl.BlockSpec((bsz, k), lambda i: (i, 0)),
                  pl.BlockSpec((k, ko), lambda i: (0, 0))],
        out_specs=pl.BlockSpec((bsz, ko), lambda i: (i, 0)),
        out_shape=jax.ShapeDtypeStruct((m, ko), jnp.float32),
    )(x, w)


def _tables_body(h_ref, d_ref, b_ref, ew_ref, ts_ref, te_ref):
    h = h_ref[...]
    ts_ref[:, :H] = jnp.dot(h, d_ref[...], preferred_element_type=jnp.float32)
    ts_ref[:, H:] = jnp.dot(h, b_ref[...], preferred_element_type=jnp.float32)
    te_ref[...] = jnp.dot(h, ew_ref[...], preferred_element_type=jnp.float32)


def _tables(h, dl, bl, ewl, blocks=10):
    bsz = NP // blocks
    return pl.pallas_call(
        _tables_body,
        grid=(blocks,),
        in_specs=[pl.BlockSpec((bsz, H), lambda i: (i, 0)),
                  pl.BlockSpec((H, H), lambda i: (0, 0)),
                  pl.BlockSpec((H, H), lambda i: (0, 0)),
                  pl.BlockSpec((H, H), lambda i: (0, 0))],
        out_specs=[
            pl.BlockSpec((bsz, 2 * H), lambda i: (i, 0)),
            pl.BlockSpec((bsz, H), lambda i: (i, 0)),
        ],
        out_shape=[jax.ShapeDtypeStruct((NP, 2 * H), jnp.float32),
                   jax.ShapeDtypeStruct((NP, H), jnp.float32)],
    )(h, dl, bl, ewl)


def _hupd_body(h_ref, a_ref, agg_ref, den_ref, o_ref):
    h = h_ref[...]
    ha = jnp.dot(h, a_ref[...], preferred_element_type=jnp.float32)
    agg = agg_ref[0, 0] + agg_ref[0, 1]
    den = den_ref[0, 0] + den_ref[0, 1]
    o_ref[...] = jnp.maximum(ha + agg / (den + 1e-6), 0.0) + h


def _h_update(h, al, agg4, den4):
    bsz = 512
    nb = NH // bsz
    return pl.pallas_call(
        _hupd_body,
        grid=(NP // bsz,),
        in_specs=[pl.BlockSpec((bsz, H), lambda i: (i, 0)),
                  pl.BlockSpec((H, H), lambda i: (0, 0)),
                  pl.BlockSpec((1, 2, bsz, H),
                               lambda i: (i // nb, 0, i % nb, 0)),
                  pl.BlockSpec((1, 2, bsz, H),
                               lambda i: (i // nb, 0, i % nb, 0))],
        out_specs=pl.BlockSpec((bsz, H), lambda i: (i, 0)),
        out_shape=jax.ShapeDtypeStruct((NP, H), jnp.float32),
    )(h, al, agg4, den4)


def _res_mm_body(ef_ref, eh_ref, c_ref, efn_ref, p_ref):
    efn = jnp.maximum(eh_ref[...], 0.0) + ef_ref[...]
    efn_ref[...] = efn
    p_ref[...] = jnp.dot(efn, c_ref[...], preferred_element_type=jnp.float32)


def _res_mm(ef, ehat, cl, blocks=40):
    bsz = EP // blocks
    return pl.pallas_call(
        _res_mm_body,
        grid=(blocks,),
        in_specs=[pl.BlockSpec((bsz, H), lambda i: (i, 0)),
                  pl.BlockSpec((bsz, H), lambda i: (i, 0)),
                  pl.BlockSpec((H, H), lambda i: (0, 0))],
        out_specs=[pl.BlockSpec((bsz, H), lambda i: (i, 0)),
                   pl.BlockSpec((bsz, H), lambda i: (i, 0))],
        out_shape=[jax.ShapeDtypeStruct((EP, H), jnp.float32),
                   jax.ShapeDtypeStruct((EP, H), jnp.float32)],
    )(ef, ehat, cl)


def _sc_edge_pass(src, dst, p, ts, te):
    @functools.partial(
        pl.kernel,
        out_type=[
            jax.ShapeDtypeStruct((EP, H), jnp.float32),
            jax.ShapeDtypeStruct((2, EP, H), jnp.float32),
            jax.ShapeDtypeStruct((4, 2, NR, H), jnp.float32),
        ],
        mesh=_mesh,
        scratch_types=[
            pltpu.VMEM((CE,), jnp.int32),
            pltpu.VMEM((CE,), jnp.int32),
            pltpu.VMEM((CE,), jnp.int32),
            pltpu.VMEM((CE, 2 * H), jnp.float32),
            pltpu.VMEM((CE, H), jnp.float32),
            pltpu.VMEM((CE, H), jnp.float32),
            pltpu.VMEM((CE, H), jnp.float32),
            pltpu.VMEM_SHARED((NR, H), jnp.float32),
            pltpu.SemaphoreType.DMA,
            pltpu.SemaphoreType.DMA,
        ],
        compiler_params=_sc_params,
    )
    def k(src_hbm, dst_hbm, p_hbm, ts_hbm, te_hbm,
          ehat_hbm, val_hbm, acc_hbm,
          si_v, di_v, ci_v, ts_v, te_v, p_v, sg_v, acc_sp, sem0, sem1):
        c = lax.axis_index("c")
        s = lax.axis_index("s")
        ebase = (c * 16 + s) * (EP // 32)
        zb = s * STRIPE

        def zero_acc():
            @pl.loop(0, CE)
            def _(r):
                @pl.loop(0, H, step=16)
                def _(j):
                    te_v[r, pl.ds(j, 16)] = jnp.zeros((16,), jnp.float32)

            pltpu.sync_copy(te_v, acc_sp.at[pl.ds(zb, CE)])
            pltpu.sync_copy(te_v, acc_sp.at[pl.ds(zb + CE, CE)])
            pltpu.sync_copy(te_v.at[pl.ds(0, STRIPE - 2 * CE)],
                            acc_sp.at[pl.ds(zb + 2 * CE, STRIPE - 2 * CE)])
            plsc.subcore_barrier()

        def readout(dst_slot):
            plsc.subcore_barrier()
            pltpu.sync_copy(acc_sp.at[pl.ds(zb, STRIPE)],
                            acc_hbm.at[dst_slot, c, pl.ds(zb, STRIPE)])
            plsc.subcore_barrier()

        zero_acc()

        @pl.loop(0, NCHUNK)
        def _(kk):
            base = ebase + kk * CE
            d1 = pltpu.async_copy(dst_hbm.at[pl.ds(base, CE)], di_v, sem0)
            d2 = pltpu.async_copy(src_hbm.at[pl.ds(base, CE)], si_v, sem0)
            d1.wait()
            d2.wait()
            g1 = pltpu.async_copy(ts_hbm.at[si_v], ts_v, sem1)
            g2 = pltpu.async_copy(te_hbm.at[di_v], te_v, sem1)
            g3 = pltpu.async_copy(p_hbm.at[pl.ds(base, CE)], p_v, sem1)
            g1.wait()
            g2.wait()
            g3.wait()

            @pl.loop(0, CE)
            def _(r):
                @pl.loop(0, H, step=16)
                def _(j):
                    dsl = pl.ds(j, 16)
                    eh = p_v[r, dsl] + ts_v[r, dsl] + te_v[r, dsl]
                    sg = 1.0 / (1.0 + jnp.exp(-eh))
                    sg_v[r, dsl] = sg * ts_v[r, pl.ds(H + j, 16)]
                    ts_v[r, pl.ds(H + j, 16)] = sg
                    te_v[r, dsl] = eh

            @pl.loop(0, CE, step=16)
            def _(i):
                d = di_v[pl.ds(i, 16)]
                ci_v[pl.ds(i, 16)] = jnp.where(d < NH, d, NH)

            w1 = pltpu.async_copy(sg_v, val_hbm.at[0, pl.ds(base, CE)], sem0)
            w2 = pltpu.async_copy(ts_v.at[:, pl.ds(H, H)],
                                  val_hbm.at[1, pl.ds(base, CE)], sem0)
            w3 = pltpu.async_copy(te_v, ehat_hbm.at[pl.ds(base, CE)], sem1)
            pltpu.sync_copy(sg_v, acc_sp.at[ci_v], add=True)
            w1.wait()
            w2.wait()
            w3.wait()

        readout(0)

        @pl.loop(0, 3)
        def _(ph):
            q = jnp.where(ph == 0, 0, 1)
            lo = jnp.where(ph == 1, 0, NH)
            zero_acc()

            @pl.loop(0, NCHUNK)
            def _(kk):
                base = ebase + kk * CE
                d1 = pltpu.async_copy(dst_hbm.at[pl.ds(base, CE)], di_v,
                                      sem0)
                d2 = pltpu.async_copy(val_hbm.at[q, pl.ds(base, CE)], sg_v,
                                      sem1)
                d1.wait()

                @pl.loop(0, CE, step=16)
                def _(i):
                    d = di_v[pl.ds(i, 16)] - lo
                    ok = (d >= 0) & (d < NH)
                    ci_v[pl.ds(i, 16)] = jnp.where(ok, d, NH)

                d2.wait()
                pltpu.sync_copy(sg_v, acc_sp.at[ci_v], add=True)

            readout(ph + 1)

    return k(src, dst, p, ts, te)


def _sc_predict(s2, d2, q, u, v):
    @functools.partial(
        pl.kernel,
        out_type=jax.ShapeDtypeStruct((ESP,), jnp.float32),
        mesh=_mesh,
        scratch_types=[
            pltpu.VMEM((NP,), jnp.float32),
            pltpu.VMEM((NP,), jnp.float32),
            pltpu.VMEM((PE,), jnp.int32),
            pltpu.VMEM((PE,), jnp.int32),
            pltpu.VMEM((PE,), jnp.float32),
        ],
        compiler_params=_sc_params,
    )
    def k(s2_hbm, d2_hbm, q_hbm, u_hbm, v_hbm, o_hbm,
          u_v, v_v, si_v, di_v, q_v):
        c = lax.axis_index("c")
        s = lax.axis_index("s")
        base = (c * 16 + s) * PE
        pltpu.sync_copy(u_hbm, u_v)
        pltpu.sync_copy(v_hbm, v_v)
        pltpu.sync_copy(s2_hbm.at[pl.ds(base, PE)], si_v)
        pltpu.sync_copy(d2_hbm.at[pl.ds(base, PE)], di_v)
        pltpu.sync_copy(q_hbm.at[pl.ds(base, PE)], q_v)

        @pl.loop(0, PE, step=16)
        def _(i):
            dsl = pl.ds(i, 16)
            ug = plsc.load_gather(u_v, [si_v[dsl]])
            vg = plsc.load_gather(v_v, [di_v[dsl]])
            q_v[dsl] = q_v[dsl] + ug + vg

        pltpu.sync_copy(q_v, o_hbm.at[pl.ds(base, PE)])

    return k(s2, d2, q, u, v)


def kernel(edge_index_sub, edge_index, x, e, e_subgraph,
           Wn, bn, We, be, A, B, C, D, Ew, Wp, bp):
    f32 = jnp.float32
    src = jnp.pad(edge_index[0].astype(jnp.int32), (0, EP - E))
    dst = jnp.pad(edge_index[1].astype(jnp.int32), (0, EP - E),
                  constant_values=N)
    s2 = jnp.pad(edge_index_sub[0].astype(jnp.int32), (0, ESP - ES))
    d2 = jnp.pad(edge_index_sub[1].astype(jnp.int32), (0, ESP - ES))

    xp = jnp.pad(x.astype(f32), ((0, NP - N), (0, 0)))
    ep = jnp.pad(e.astype(f32), ((0, EP - E), (0, 0)))
    esp = jnp.pad(e_subgraph.astype(f32), ((0, ESP - ES), (0, 0)))

    h = _mm_bias(xp, Wn, bn.reshape(1, H), blocks=10)
    ef = _mm_bias(ep, We, be.reshape(1, H), blocks=40)

    for l in range(A.shape[0]):
        ts, te = _tables(h, D[l], B[l], Ew[l])
        if l == 0:
            p = _mm(ef, C[0], blocks=40)
        else:
            ef, p = _res_mm(ef, ehat, C[l])
        ehat, _, acc = _sc_edge_pass(src, dst, p, ts, te)
        h = _h_update(h, A[l], acc[:2], acc[2:])

    wp12 = jnp.concatenate([Wp[:H], Wp[H:2 * H]], axis=1)
    w3 = We @ Wp[2 * H:]
    cst = (be @ Wp[2 * H:] + bp).reshape(1, 1)

    uv = _mm(h, wp12, blocks=10)
    q = _mm_bias(esp, w3, cst, blocks=25).reshape(ESP)
    scores = _sc_predict(s2, d2, q, uv[:, 0], uv[:, 1])
    return scores[:ES].reshape(ES, 1)


# speedup vs baseline: 1.3479x; 1.0159x over previous
import functools

import jax
import jax.numpy as jnp
from jax import lax
from jax.experimental import pallas as pl
from jax.experimental.pallas import tpu as pltpu
from jax.experimental.pallas import tpu_sc as plsc

N = 10000
NP = 10240
E = 320000
EP = 327680
ES = 100000
ESP = 102400
H = 128
HH = 64
DE = 16
CE = 128
NCHUNK = EP // 32 // CE
NH = 5120
NR = 5632
STRIPE = NR // 16
PE = ESP // 32

_mesh = plsc.VectorSubcoreMesh(core_axis_name="c", subcore_axis_name="s")
_sc_params = pltpu.CompilerParams(needs_layout_passes=False)


def _mm_bias_body(x_ref, w_ref, b_ref, o_ref):
    o_ref[...] = jnp.dot(x_ref[...], w_ref[...],
                         preferred_element_type=jnp.float32) + b_ref[...]


def _mm_bias(x, w, b, blocks):
    m, k = x.shape
    ko = w.shape[1]
    bsz = m // blocks
    return pl.pallas_call(
        _mm_bias_body,
        grid=(blocks,),
        in_specs=[pl.BlockSpec((bsz, k), lambda i: (i, 0)),
                  pl.BlockSpec((k, ko), lambda i: (0, 0)),
                  pl.BlockSpec((1, ko), lambda i: (0, 0))],
        out_specs=pl.BlockSpec((bsz, ko), lambda i: (i, 0)),
        out_shape=jax.ShapeDtypeStruct((m, ko), jnp.float32),
    )(x, w, b)


def _mm_body(x_ref, w_ref, o_ref):
    o_ref[...] = jnp.dot(x_ref[...], w_ref[...],
                         preferred_element_type=jnp.float32)


def _mm(x, w, blocks):
    m, k = x.shape
    ko = w.shape[1]
    bsz = m // blocks
    return pl.pallas_call(
        _mm_body,
        grid=(blocks,),
        in_specs=[pl.BlockSpec((bsz, k), lambda i: (i, 0)),
                  pl.BlockSpec((k, ko), lambda i: (0, 0))],
        out_specs=pl.BlockSpec((bsz, ko), lambda i: (i, 0)),
        out_shape=jax.ShapeDtypeStruct((m, ko), jnp.float32),
    )(x, w)


def _tables_body(h_ref, d_ref, b_ref, ew_ref, ts_ref, te_ref):
    h = h_ref[...]
    ts_ref[:, :H] = jnp.dot(h, d_ref[...], preferred_element_type=jnp.float32)
    ts_ref[:, H:] = jnp.dot(h, b_ref[...], preferred_element_type=jnp.float32)
    te_ref[...] = jnp.dot(h, ew_ref[...], preferred_element_type=jnp.float32)


def _tables(h, dl, bl, ewl, blocks=10):
    bsz = NP // blocks
    return pl.pallas_call(
        _tables_body,
        grid=(blocks,),
        in_specs=[pl.BlockSpec((bsz, H), lambda i: (i, 0)),
                  pl.BlockSpec((H, H), lambda i: (0, 0)),
                  pl.BlockSpec((H, H), lambda i: (0, 0)),
                  pl.BlockSpec((H, H), lambda i: (0, 0))],
        out_specs=[
            pl.BlockSpec((bsz, 2 * H), lambda i: (i, 0)),
            pl.BlockSpec((bsz, H), lambda i: (i, 0)),
        ],
        out_shape=[jax.ShapeDtypeStruct((NP, 2 * H), jnp.float32),
                   jax.ShapeDtypeStruct((NP, H), jnp.float32)],
    )(h, dl, bl, ewl)


def _hupd_body(h_ref, a_ref, agg_ref, den_ref, o_ref):
    h = h_ref[...]
    ha = jnp.dot(h, a_ref[...], preferred_element_type=jnp.float32)
    agg = agg_ref[0, 0] + agg_ref[0, 1]
    den = den_ref[0, 0] + den_ref[0, 1]
    o_ref[...] = jnp.maximum(ha + agg / (den + 1e-6), 0.0) + h


def _h_update(h, al, agg4, den4):
    bsz = 512
    nb = NH // bsz
    return pl.pallas_call(
        _hupd_body,
        grid=(NP // bsz,),
        in_specs=[pl.BlockSpec((bsz, H), lambda i: (i, 0)),
                  pl.BlockSpec((H, H), lambda i: (0, 0)),
                  pl.BlockSpec((1, 2, bsz, H),
                               lambda i: (i // nb, 0, i % nb, 0)),
                  pl.BlockSpec((1, 2, bsz, H),
                               lambda i: (i // nb, 0, i % nb, 0))],
        out_specs=pl.BlockSpec((bsz, H), lambda i: (i, 0)),
        out_shape=jax.ShapeDtypeStruct((NP, H), jnp.float32),
    )(h, al, agg4, den4)


def _res_mm_body(ef_ref, eh_ref, c_ref, efn_ref, p_ref):
    efn = jnp.maximum(eh_ref[...], 0.0) + ef_ref[...]
    efn_ref[...] = efn
    p_ref[...] = jnp.dot(efn, c_ref[...], preferred_element_type=jnp.float32)


def _res_mm(ef, ehat, cl, blocks=40):
    bsz = EP // blocks
    return pl.pallas_call(
        _res_mm_body,
        grid=(blocks,),
        in_specs=[pl.BlockSpec((bsz, H), lambda i: (i, 0)),
                  pl.BlockSpec((bsz, H), lambda i: (i, 0)),
                  pl.BlockSpec((H, H), lambda i: (0, 0))],
        out_specs=[pl.BlockSpec((bsz, H), lambda i: (i, 0)),
                   pl.BlockSpec((bsz, H), lambda i: (i, 0))],
        out_shape=[jax.ShapeDtypeStruct((EP, H), jnp.float32),
                   jax.ShapeDtypeStruct((EP, H), jnp.float32)],
    )(ef, ehat, cl)


def _sc_edge_pass(src, dst, p, ts, te):
    @functools.partial(
        pl.kernel,
        out_type=[
            jax.ShapeDtypeStruct((EP, H), jnp.float32),
            jax.ShapeDtypeStruct((2, EP, H), jnp.float32),
            jax.ShapeDtypeStruct((4, 2, NR, H), jnp.float32),
        ],
        mesh=_mesh,
        scratch_types=[
            pltpu.VMEM((CE,), jnp.int32),
            pltpu.VMEM((CE,), jnp.int32),
            pltpu.VMEM((CE,), jnp.int32),
            pltpu.VMEM((CE,), jnp.int32),
            pltpu.VMEM((CE, 2 * H), jnp.float32),
            pltpu.VMEM((CE, H), jnp.float32),
            pltpu.VMEM((CE, H), jnp.float32),
            pltpu.VMEM((CE, H), jnp.float32),
            pltpu.VMEM_SHARED((NR, H), jnp.float32),
            pltpu.SemaphoreType.DMA,
            pltpu.SemaphoreType.DMA,
        ],
        compiler_params=_sc_params,
    )
    def k(src_hbm, dst_hbm, p_hbm, ts_hbm, te_hbm,
          ehat_hbm, val_hbm, acc_hbm,
          si_v, di_v, ci_v, ci2_v, ts_v, te_v, p_v, sg_v, acc_sp,
          sem0, sem1):
        c = lax.axis_index("c")
        s = lax.axis_index("s")
        ebase = (c * 16 + s) * (EP // 32)
        zb = s * STRIPE

        def zero_acc():
            @pl.loop(0, CE)
            def _(r):
                @pl.loop(0, H, step=16)
                def _(j):
                    te_v[r, pl.ds(j, 16)] = jnp.zeros((16,), jnp.float32)

            pltpu.sync_copy(te_v, acc_sp.at[pl.ds(zb, CE)])
            pltpu.sync_copy(te_v, acc_sp.at[pl.ds(zb + CE, CE)])
            pltpu.sync_copy(te_v.at[pl.ds(0, STRIPE - 2 * CE)],
                            acc_sp.at[pl.ds(zb + 2 * CE, STRIPE - 2 * CE)])
            plsc.subcore_barrier()

        def readout(dst_slot):
            plsc.subcore_barrier()
            pltpu.sync_copy(acc_sp.at[pl.ds(zb, STRIPE)],
                            acc_hbm.at[dst_slot, c, pl.ds(zb, STRIPE)])
            plsc.subcore_barrier()

        zero_acc()

        @pl.loop(0, NCHUNK)
        def _(kk):
            base = ebase + kk * CE
            d1 = pltpu.async_copy(dst_hbm.at[pl.ds(base, CE)], di_v, sem0)
            d2 = pltpu.async_copy(src_hbm.at[pl.ds(base, CE)], si_v, sem0)
            d1.wait()
            d2.wait()
            g1 = pltpu.async_copy(ts_hbm.at[si_v], ts_v, sem1)
            g2 = pltpu.async_copy(te_hbm.at[di_v], te_v, sem1)
            g3 = pltpu.async_copy(p_hbm.at[pl.ds(base, CE)], p_v, sem1)
            g1.wait()
            g2.wait()
            g3.wait()

            @pl.loop(0, CE)
            def _(r):
                @pl.loop(0, H, step=16)
                def _(j):
                    dsl = pl.ds(j, 16)
                    eh = p_v[r, dsl] + ts_v[r, dsl] + te_v[r, dsl]
                    sg = 1.0 / (1.0 + jnp.exp(-eh))
                    sg_v[r, dsl] = sg * ts_v[r, pl.ds(H + j, 16)]
                    ts_v[r, pl.ds(H + j, 16)] = sg
                    te_v[r, dsl] = eh

            @pl.loop(0, CE, step=16)
            def _(i):
                d = di_v[pl.ds(i, 16)]
                ci_v[pl.ds(i, 16)] = jnp.where(d < NH, d, NH)

            w1 = pltpu.async_copy(sg_v, val_hbm.at[0, pl.ds(base, CE)], sem0)
            w2 = pltpu.async_copy(ts_v.at[:, pl.ds(H, H)],
                                  val_hbm.at[1, pl.ds(base, CE)], sem0)
            w3 = pltpu.async_copy(te_v, ehat_hbm.at[pl.ds(base, CE)], sem1)
            pltpu.sync_copy(sg_v, acc_sp.at[ci_v], add=True)
            w1.wait()
            w2.wait()
            w3.wait()

        readout(0)

        @pl.loop(0, 3)
        def _(ph):
            q = jnp.where(ph == 0, 0, 1)
            lo = jnp.where(ph == 1, 0, NH)
            zero_acc()

            def clamp(idx_ref, out_ref):
                @pl.loop(0, CE, step=16)
                def _(i):
                    d = idx_ref[pl.ds(i, 16)] - lo
                    ok = (d >= 0) & (d < NH)
                    out_ref[pl.ds(i, 16)] = jnp.where(ok, d, NH)

            @pl.loop(0, NCHUNK, step=2)
            def _(kk):
                base = ebase + kk * CE
                a1 = pltpu.async_copy(dst_hbm.at[pl.ds(base, CE)], di_v,
                                      sem0)
                a2 = pltpu.async_copy(val_hbm.at[q, pl.ds(base, CE)], sg_v,
                                      sem1)
                a1.wait()
                clamp(di_v, ci_v)
                b1 = pltpu.async_copy(dst_hbm.at[pl.ds(base + CE, CE)],
                                      si_v, sem0)
                b2 = pltpu.async_copy(val_hbm.at[q, pl.ds(base + CE, CE)],
                                      p_v, sem1)
                a2.wait()
                pltpu.sync_copy(sg_v, acc_sp.at[ci_v], add=True)
                b1.wait()
                clamp(si_v, ci2_v)
                b2.wait()
                pltpu.sync_copy(p_v, acc_sp.at[ci2_v], add=True)

            readout(ph + 1)

    return k(src, dst, p, ts, te)


def _sc_predict(s2, d2, q, u, v):
    @functools.partial(
        pl.kernel,
        out_type=jax.ShapeDtypeStruct((ESP,), jnp.float32),
        mesh=_mesh,
        scratch_types=[
            pltpu.VMEM((NP,), jnp.float32),
            pltpu.VMEM((NP,), jnp.float32),
            pltpu.VMEM((PE,), jnp.int32),
            pltpu.VMEM((PE,), jnp.int32),
            pltpu.VMEM((PE,), jnp.float32),
        ],
        compiler_params=_sc_params,
    )
    def k(s2_hbm, d2_hbm, q_hbm, u_hbm, v_hbm, o_hbm,
          u_v, v_v, si_v, di_v, q_v):
        c = lax.axis_index("c")
        s = lax.axis_index("s")
        base = (c * 16 + s) * PE
        pltpu.sync_copy(u_hbm, u_v)
        pltpu.sync_copy(v_hbm, v_v)
        pltpu.sync_copy(s2_hbm.at[pl.ds(base, PE)], si_v)
        pltpu.sync_copy(d2_hbm.at[pl.ds(base, PE)], di_v)
        pltpu.sync_copy(q_hbm.at[pl.ds(base, PE)], q_v)

        @pl.loop(0, PE, step=16)
        def _(i):
            dsl = pl.ds(i, 16)
            ug = plsc.load_gather(u_v, [si_v[dsl]])
            vg = plsc.load_gather(v_v, [di_v[dsl]])
            q_v[dsl] = q_v[dsl] + ug + vg

        pltpu.sync_copy(q_v, o_hbm.at[pl.ds(base, PE)])

    return k(s2, d2, q, u, v)


def kernel(edge_index_sub, edge_index, x, e, e_subgraph,
           Wn, bn, We, be, A, B, C, D, Ew, Wp, bp):
    f32 = jnp.float32
    src = jnp.pad(edge_index[0].astype(jnp.int32), (0, EP - E))
    dst = jnp.pad(edge_index[1].astype(jnp.int32), (0, EP - E),
                  constant_values=N)
    s2 = jnp.pad(edge_index_sub[0].astype(jnp.int32), (0, ESP - ES))
    d2 = jnp.pad(edge_index_sub[1].astype(jnp.int32), (0, ESP - ES))

    xp = jnp.pad(x.astype(f32), ((0, NP - N), (0, 0)))
    ep = jnp.pad(e.astype(f32), ((0, EP - E), (0, 0)))
    esp = jnp.pad(e_subgraph.astype(f32), ((0, ESP - ES), (0, 0)))

    h = _mm_bias(xp, Wn, bn.reshape(1, H), blocks=10)
    ef = _mm_bias(ep, We, be.reshape(1, H), blocks=40)

    for l in range(A.shape[0]):
        ts, te = _tables(h, D[l], B[l], Ew[l])
        if l == 0:
            p = _mm(ef, C[0], blocks=40)
        else:
            ef, p = _res_mm(ef, ehat, C[l])
        ehat, _, acc = _sc_edge_pass(src, dst, p, ts, te)
        h = _h_update(h, A[l], acc[:2], acc[2:])

    wp12 = jnp.concatenate([Wp[:H], Wp[H:2 * H]], axis=1)
    w3 = We @ Wp[2 * H:]
    cst = (be @ Wp[2 * H:] + bp).reshape(1, 1)

    uv = _mm(h, wp12, blocks=10)
    q = _mm_bias(esp, w3, cst, blocks=25).reshape(ESP)
    scores = _sc_predict(s2, d2, q, uv[:, 0], uv[:, 1])
    return scores[:ES].reshape(ES, 1)
